# Initial kernel scaffold; baseline (speedup 1.0000x reference)
import functools

import jax
import jax.numpy as jnp
from jax import lax
from jax.experimental import pallas as pl
from jax.experimental.pallas import tpu as pltpu
from jax.experimental.pallas import tpu_sc as plsc

N = 10000
E = 320000
NF = 128
NCLS = 40

NUM_SC = 2
NUM_SUB = 16
NW = NUM_SC * NUM_SUB

NPAD = 10016
CPAD = 48
BLK = 128
EB = 80
EPW = EB * BLK
EPAD = NW * EPW
RPS = NPAD // NUM_SUB

_mesh = plsc.VectorSubcoreMesh(core_axis_name="c", subcore_axis_name="s")


def _hist_body(dst_hbm, out_hbm, idx_v, hist_v):
    wid = lax.axis_index("s") * NUM_SC + lax.axis_index("c")
    pltpu.sync_copy(dst_hbm.at[wid], idx_v)
    zeros16 = jnp.zeros((16,), jnp.float32)

    @pl.loop(0, NPAD // 16)
    def _(i):
        hist_v[pl.ds(i * 16, 16)] = zeros16

    ones16 = jnp.ones((16,), jnp.float32)

    @pl.loop(0, EPW // 16)
    def _(i):
        idx = idx_v[pl.ds(i * 16, 16)]
        plsc.addupdate_scatter(hist_v, [idx], ones16)

    pltpu.sync_copy(hist_v, out_hbm.at[wid])


_hist_call = functools.partial(
    pl.kernel,
    out_type=jax.ShapeDtypeStruct((NW, NPAD), jnp.float32),
    mesh=_mesh,
    scratch_types=[
        pltpu.VMEM((EPW,), jnp.int32),
        pltpu.VMEM((NPAD,), jnp.float32),
    ],
)(_hist_body)


def _g_body(x_ref, w_ref, hist_ref, g_ref):
    ones = jnp.ones((NW, 1), jnp.float32)
    deg = lax.dot_general(
        hist_ref[...], ones,
        dimension_numbers=(((0,), (0,)), ((), ())),
        preferred_element_type=jnp.float32,
    ) + 1.0
    dinv = lax.rsqrt(deg)
    h = jnp.dot(x_ref[...], w_ref[...], preferred_element_type=jnp.float32)
    g_ref[...] = h * dinv


def _g_call(x_pad, w_pad, hist):
    return pl.pallas_call(
        _g_body,
        out_shape=jax.ShapeDtypeStruct((NPAD, CPAD), jnp.float32),
    )(x_pad, w_pad, hist)


def _main_body(g_hbm, src_hbm, dst_hbm, zero_hbm, out_hbm,
               src_v, dst_v, rows_v, acc_sh, sem):
    c = lax.axis_index("c")
    s = lax.axis_index("s")
    wid = s * NUM_SC + c
    pltpu.sync_copy(zero_hbm.at[pl.ds(s * RPS, RPS)],
                    acc_sh.at[pl.ds(s * RPS, RPS)])
    pltpu.sync_copy(src_hbm.at[wid], src_v)
    pltpu.sync_copy(dst_hbm.at[wid], dst_v)
    plsc.subcore_barrier()

    @pl.loop(0, EB)
    def _(j):
        pltpu.async_copy(g_hbm.at[src_v.at[j]], rows_v, sem).wait()
        pltpu.sync_copy(rows_v, acc_sh.at[dst_v.at[j]], add=True)

    plsc.subcore_barrier()
    pltpu.sync_copy(acc_sh.at[pl.ds(s * RPS, RPS)],
                    out_hbm.at[c, pl.ds(s * RPS, RPS)])


_main_call = functools.partial(
    pl.kernel,
    out_type=jax.ShapeDtypeStruct((NUM_SC, NPAD, CPAD), jnp.float32),
    mesh=_mesh,
    scratch_types=[
        pltpu.VMEM((EB, BLK), jnp.int32),
        pltpu.VMEM((EB, BLK), jnp.int32),
        pltpu.VMEM((BLK, CPAD), jnp.float32),
        pltpu.VMEM_SHARED((NPAD, CPAD), jnp.float32),
        pltpu.SemaphoreType.DMA,
    ],
)(_main_body)


def _final_body(acc_ref, g_ref, hist_ref, b_ref, o_ref):
    ones = jnp.ones((NW, 1), jnp.float32)
    deg = lax.dot_general(
        hist_ref[...], ones,
        dimension_numbers=(((0,), (0,)), ((), ())),
        preferred_element_type=jnp.float32,
    ) + 1.0
    dinv = lax.rsqrt(deg)
    total = acc_ref[0] + acc_ref[1] + g_ref[...]
    z = total * dinv + b_ref[...]
    m = jnp.max(z, axis=1, keepdims=True)
    lse = jnp.log(jnp.sum(jnp.exp(z - m), axis=1, keepdims=True))
    o_ref[...] = (z - m - lse)[:N, :NCLS]


def _final_call(acc, g, hist, b_pad):
    return pl.pallas_call(
        _final_body,
        out_shape=jax.ShapeDtypeStruct((N, NCLS), jnp.float32),
    )(acc, g, hist, b_pad)


@jax.jit
def kernel(x, edge_index, W, b):
    src = edge_index[0].astype(jnp.int32)
    dst = edge_index[1].astype(jnp.int32)
    pad = jnp.full((EPAD - E,), N, jnp.int32)
    src_p = jnp.concatenate([src, pad]).reshape(NW, EB, BLK)
    dst_p = jnp.concatenate([dst, pad]).reshape(NW, EB, BLK)
    dst_flat = dst_p.reshape(NW, EPW)

    x_pad = jnp.pad(x.astype(jnp.float32), ((0, NPAD - N), (0, 0)))
    w_pad = jnp.pad(W.astype(jnp.float32), ((0, 0), (0, CPAD - NCLS)))
    b_pad = jnp.full((1, CPAD), -1e30, jnp.float32).at[0, :NCLS].set(b)
    zeros = jnp.zeros((NPAD, CPAD), jnp.float32)

    hist = _hist_call(dst_flat)
    g = _g_call(x_pad, w_pad, hist)
    acc = _main_call(g, src_p, dst_p, zeros)
    return _final_call(acc, g, hist, b_pad)


# speedup vs baseline: 27.0359x; 27.0359x over previous
import dataclasses
import functools

import jax
import jax.numpy as jnp
from jax import lax
from jax.experimental import pallas as pl
from jax.experimental.pallas import tpu as pltpu
from jax.experimental.pallas import tpu_sc as plsc

N = 10000
E = 320000
NF = 128
NCLS = 40

NUM_SC = 2
NUM_SUB = 16
NW = NUM_SC * NUM_SUB

NPAD = 10112
CPAD = 48
BLK = 128
EB = 80
EPW = EB * BLK
EPAD = NW * EPW
RPS = NPAD // NUM_SUB

_mesh = plsc.VectorSubcoreMesh(
    core_axis_name="c", subcore_axis_name="s",
    num_cores=NUM_SC, num_subcores=NUM_SUB,
)


def _hist_body(dst_hbm, out_hbm, idx_v, hist_v):
    wid = lax.axis_index("s") * NUM_SC + lax.axis_index("c")
    pltpu.sync_copy(dst_hbm.at[wid], idx_v)
    zeros16 = jnp.zeros((16,), jnp.float32)

    @pl.loop(0, NPAD // 16)
    def _(i):
        hist_v[pl.ds(i * 16, 16)] = zeros16

    ones16 = jnp.ones((16,), jnp.float32)

    @pl.loop(0, EPW // 16)
    def _(i):
        idx = idx_v[pl.ds(i * 16, 16)]
        plsc.addupdate_scatter(hist_v, [idx], ones16)

    pltpu.sync_copy(hist_v, out_hbm.at[wid])


_sc_params = pltpu.CompilerParams(
    needs_layout_passes=False, use_tc_tiling_on_sc=False
)

_hist_call = functools.partial(
    pl.kernel,
    out_type=jax.ShapeDtypeStruct((NW, NPAD), jnp.float32),
    mesh=_mesh,
    compiler_params=_sc_params,
    scratch_types=[
        pltpu.VMEM((EPW,), jnp.int32),
        pltpu.VMEM((NPAD,), jnp.float32),
    ],
)(_hist_body)


def _g_body(x_ref, w_ref, hist_ref, g_ref):
    ones = jnp.ones((NW, 1), jnp.float32)
    deg = lax.dot_general(
        hist_ref[...], ones,
        dimension_numbers=(((0,), (0,)), ((), ())),
        preferred_element_type=jnp.float32,
    ) + 1.0
    dinv = lax.rsqrt(deg)
    h = jnp.dot(x_ref[...], w_ref[...], preferred_element_type=jnp.float32)
    g_ref[...] = h * dinv


def _g_call(x_pad, w_pad, hist):
    return pl.pallas_call(
        _g_body,
        out_shape=jax.ShapeDtypeStruct((NPAD, CPAD), jnp.float32),
    )(x_pad, w_pad, hist)


def _main_body(g_hbm, src_hbm, dst_hbm, zero_hbm, out_hbm,
               src_v, dst_v, rows_v, acc_sh, sem):
    c = lax.axis_index("c")
    s = lax.axis_index("s")
    wid = s * NUM_SC + c
    pltpu.sync_copy(zero_hbm.at[pl.ds(s * RPS, RPS)],
                    acc_sh.at[pl.ds(s * RPS, RPS)])
    pltpu.sync_copy(src_hbm.at[wid], src_v)
    pltpu.sync_copy(dst_hbm.at[wid], dst_v)
    plsc.subcore_barrier()

    @pl.loop(0, EB)
    def _(j):
        pltpu.async_copy(g_hbm.at[src_v.at[j]], rows_v, sem).wait()
        pltpu.sync_copy(rows_v, acc_sh.at[dst_v.at[j]], add=True)

    plsc.subcore_barrier()
    pltpu.sync_copy(acc_sh.at[pl.ds(s * RPS, RPS)],
                    out_hbm.at[c, pl.ds(s * RPS, RPS)])


_main_call = functools.partial(
    pl.kernel,
    out_type=jax.ShapeDtypeStruct((NUM_SC, NPAD, CPAD), jnp.float32),
    mesh=_mesh,
    compiler_params=_sc_params,
    scratch_types=[
        pltpu.VMEM((EB, BLK), jnp.int32),
        pltpu.VMEM((EB, BLK), jnp.int32),
        pltpu.VMEM((BLK, CPAD), jnp.float32),
        pltpu.VMEM_SHARED((NPAD, CPAD), jnp.float32),
        pltpu.SemaphoreType.DMA,
    ],
)(_main_body)


def _final_body(acc_ref, g_ref, hist_ref, b_ref, o_ref):
    ones = jnp.ones((NW, 1), jnp.float32)
    deg = lax.dot_general(
        hist_ref[...], ones,
        dimension_numbers=(((0,), (0,)), ((), ())),
        preferred_element_type=jnp.float32,
    ) + 1.0
    dinv = lax.rsqrt(deg)
    total = acc_ref[0] + acc_ref[1] + g_ref[...]
    z = total * dinv + b_ref[...]
    m = jnp.max(z, axis=1, keepdims=True)
    lse = jnp.log(jnp.sum(jnp.exp(z - m), axis=1, keepdims=True))
    o_ref[...] = (z - m - lse)[:N, :NCLS]


def _final_call(acc, g, hist, b_pad):
    return pl.pallas_call(
        _final_body,
        out_shape=jax.ShapeDtypeStruct((N, NCLS), jnp.float32),
    )(acc, g, hist, b_pad)


@jax.jit
def kernel(x, edge_index, W, b):
    src = edge_index[0].astype(jnp.int32)
    dst = edge_index[1].astype(jnp.int32)
    pad = jnp.full((EPAD - E,), N, jnp.int32)
    src_p = jnp.concatenate([src, pad]).reshape(NW, EB, BLK)
    dst_p = jnp.concatenate([dst, pad]).reshape(NW, EB, BLK)
    dst_flat = dst_p.reshape(NW, EPW)

    x_pad = jnp.pad(x.astype(jnp.float32), ((0, NPAD - N), (0, 0)))
    w_pad = jnp.pad(W.astype(jnp.float32), ((0, 0), (0, CPAD - NCLS)))
    b_pad = jnp.full((1, CPAD), -1e30, jnp.float32).at[0, :NCLS].set(b)
    zeros = jnp.zeros((NPAD, CPAD), jnp.float32)

    hist = _hist_call(dst_flat)
    g = _g_call(x_pad, w_pad, hist)
    acc = _main_call(g, src_p, dst_p, zeros)
    return _final_call(acc, g, hist, b_pad)


# speedup vs baseline: 29.1251x; 1.0773x over previous
import dataclasses
import functools

import jax
import jax.numpy as jnp
from jax import lax
from jax.experimental import pallas as pl
from jax.experimental.pallas import tpu as pltpu
from jax.experimental.pallas import tpu_sc as plsc

N = 10000
E = 320000
NF = 128
NCLS = 40

NUM_SC = 2
NUM_SUB = 16
NW = NUM_SC * NUM_SUB

NPAD = 10112
CPAD = 48
BLK = 128
EB = 80
EPW = EB * BLK
EPAD = NW * EPW
RPS = NPAD // NUM_SUB

_mesh = plsc.VectorSubcoreMesh(
    core_axis_name="c", subcore_axis_name="s",
    num_cores=NUM_SC, num_subcores=NUM_SUB,
)


def _hist_body(dst_hbm, out_hbm, idx_v, hist_v):
    wid = lax.axis_index("s") * NUM_SC + lax.axis_index("c")
    pltpu.sync_copy(dst_hbm.at[wid], idx_v)
    zeros16 = jnp.zeros((16,), jnp.float32)

    @pl.loop(0, NPAD // 16)
    def _(i):
        hist_v[pl.ds(i * 16, 16)] = zeros16

    ones16 = jnp.ones((16,), jnp.float32)

    @pl.loop(0, EPW // 16)
    def _(i):
        idx = idx_v[pl.ds(i * 16, 16)]
        plsc.addupdate_scatter(hist_v, [idx], ones16)

    pltpu.sync_copy(hist_v, out_hbm.at[wid])


_sc_params = pltpu.CompilerParams(
    needs_layout_passes=False, use_tc_tiling_on_sc=False
)

_hist_call = functools.partial(
    pl.kernel,
    out_type=jax.ShapeDtypeStruct((NW, NPAD), jnp.float32),
    mesh=_mesh,
    compiler_params=_sc_params,
    scratch_types=[
        pltpu.VMEM((EPW,), jnp.int32),
        pltpu.VMEM((NPAD,), jnp.float32),
    ],
)(_hist_body)


def _h_body(x_ref, w_ref, h_ref):
    h_ref[...] = jnp.dot(
        x_ref[...], w_ref[...], preferred_element_type=jnp.float32
    )


def _h_call(x_pad, w_pad):
    return pl.pallas_call(
        _h_body,
        out_shape=jax.ShapeDtypeStruct((NPAD, CPAD), jnp.float32),
    )(x_pad, w_pad)


def _g_body(h_ref, hist_ref, g_ref):
    ones = jnp.ones((NW, 1), jnp.float32)
    deg = lax.dot_general(
        hist_ref[...], ones,
        dimension_numbers=(((0,), (0,)), ((), ())),
        preferred_element_type=jnp.float32,
    ) + 1.0
    dinv = lax.rsqrt(deg)
    g_ref[...] = h_ref[...] * dinv


def _g_call(h, hist):
    return pl.pallas_call(
        _g_body,
        out_shape=jax.ShapeDtypeStruct((NPAD, CPAD), jnp.float32),
    )(h, hist)


def _main_body(g_hbm, src_hbm, dst_hbm, zero_hbm, out_hbm,
               src_v, dst_v, rows0, rows1, acc_sh, sem0, sem1):
    c = lax.axis_index("c")
    s = lax.axis_index("s")
    wid = s * NUM_SC + c
    pltpu.sync_copy(zero_hbm.at[pl.ds(s * RPS, RPS)],
                    acc_sh.at[pl.ds(s * RPS, RPS)])
    pltpu.sync_copy(src_hbm.at[wid], src_v)
    pltpu.sync_copy(dst_hbm.at[wid], dst_v)
    plsc.subcore_barrier()

    pltpu.async_copy(g_hbm.at[src_v.at[0]], rows0, sem0)
    pltpu.async_copy(g_hbm.at[src_v.at[1]], rows1, sem1)

    @pl.loop(0, EB, step=2)
    def _(j):
        pltpu.make_async_copy(g_hbm.at[src_v.at[0]], rows0, sem0).wait()
        pltpu.sync_copy(rows0, acc_sh.at[dst_v.at[j]], add=True)

        @pl.when(j + 2 < EB)
        def _():
            pltpu.async_copy(g_hbm.at[src_v.at[j + 2]], rows0, sem0)

        pltpu.make_async_copy(g_hbm.at[src_v.at[1]], rows1, sem1).wait()
        pltpu.sync_copy(rows1, acc_sh.at[dst_v.at[j + 1]], add=True)

        @pl.when(j + 3 < EB)
        def _():
            pltpu.async_copy(g_hbm.at[src_v.at[j + 3]], rows1, sem1)

    plsc.subcore_barrier()
    pltpu.sync_copy(acc_sh.at[pl.ds(s * RPS, RPS)],
                    out_hbm.at[c, pl.ds(s * RPS, RPS)])


_main_call = functools.partial(
    pl.kernel,
    out_type=jax.ShapeDtypeStruct((NUM_SC, NPAD, CPAD), jnp.float32),
    mesh=_mesh,
    compiler_params=_sc_params,
    scratch_types=[
        pltpu.VMEM((EB, BLK), jnp.int32),
        pltpu.VMEM((EB, BLK), jnp.int32),
        pltpu.VMEM((BLK, CPAD), jnp.float32),
        pltpu.VMEM((BLK, CPAD), jnp.float32),
        pltpu.VMEM_SHARED((NPAD, CPAD), jnp.float32),
        pltpu.SemaphoreType.DMA,
        pltpu.SemaphoreType.DMA,
    ],
)(_main_body)


def _final_body(acc_ref, g_ref, hist_ref, b_ref, o_ref):
    ones = jnp.ones((NW, 1), jnp.float32)
    deg = lax.dot_general(
        hist_ref[...], ones,
        dimension_numbers=(((0,), (0,)), ((), ())),
        preferred_element_type=jnp.float32,
    ) + 1.0
    dinv = lax.rsqrt(deg)
    total = acc_ref[0] + acc_ref[1] + g_ref[...]
    z = total * dinv + b_ref[...]
    m = jnp.max(z, axis=1, keepdims=True)
    lse = jnp.log(jnp.sum(jnp.exp(z - m), axis=1, keepdims=True))
    o_ref[...] = (z - m - lse)[:N, :NCLS]


def _final_call(acc, g, hist, b_pad):
    return pl.pallas_call(
        _final_body,
        out_shape=jax.ShapeDtypeStruct((N, NCLS), jnp.float32),
    )(acc, g, hist, b_pad)


@jax.jit
def kernel(x, edge_index, W, b):
    src = edge_index[0].astype(jnp.int32)
    dst = edge_index[1].astype(jnp.int32)
    pad_src = jnp.full((EPAD - E,), N, jnp.int32)
    pad_dst = N + jnp.arange(EPAD - E, dtype=jnp.int32) % (NPAD - N)
    src_p = jnp.concatenate([src, pad_src]).reshape(NW, EB, BLK)
    dst_p = jnp.concatenate([dst, pad_dst]).reshape(NW, EB, BLK)
    dst_flat = dst_p.reshape(NW, EPW)

    x_pad = jnp.pad(x.astype(jnp.float32), ((0, NPAD - N), (0, 0)))
    w_pad = jnp.pad(W.astype(jnp.float32), ((0, 0), (0, CPAD - NCLS)))
    b_pad = jnp.full((1, CPAD), -1e30, jnp.float32).at[0, :NCLS].set(b)
    zeros = jnp.zeros((NPAD, CPAD), jnp.float32)

    hist = _hist_call(dst_flat)
    h = _h_call(x_pad, w_pad)
    g = _g_call(h, hist)
    acc = _main_call(g, src_p, dst_p, zeros)
    return _final_call(acc, g, hist, b_pad)


# speedup vs baseline: 52.1006x; 1.7889x over previous
import dataclasses
import functools

import jax
import jax.numpy as jnp
from jax import lax
from jax.experimental import pallas as pl
from jax.experimental.pallas import tpu as pltpu
from jax.experimental.pallas import tpu_sc as plsc

N = 10000
E = 320000
NF = 128
NCLS = 40

NUM_SC = 2
NUM_SUB = 16
NW = NUM_SC * NUM_SUB

NPAD = 10112
CPAD = 48
BLK = 128
EB = 80
EPW = EB * BLK
EPAD = NW * EPW
RPS = NPAD // NUM_SUB

_mesh = plsc.VectorSubcoreMesh(
    core_axis_name="c", subcore_axis_name="s",
    num_cores=NUM_SC, num_subcores=NUM_SUB,
)


def _hist_body(dst_hbm, out_hbm, idx_v, hist_v):
    wid = lax.axis_index("s") * NUM_SC + lax.axis_index("c")
    pltpu.sync_copy(dst_hbm.at[wid], idx_v)
    zeros16 = jnp.zeros((16,), jnp.float32)

    @pl.loop(0, NPAD // 16)
    def _(i):
        hist_v[pl.ds(i * 16, 16)] = zeros16

    ones16 = jnp.ones((16,), jnp.float32)

    @pl.loop(0, EPW // 16)
    def _(i):
        idx = idx_v[pl.ds(i * 16, 16)]
        plsc.addupdate_scatter(hist_v, [idx], ones16)

    pltpu.sync_copy(hist_v, out_hbm.at[wid])


_sc_params = pltpu.CompilerParams(
    needs_layout_passes=False, use_tc_tiling_on_sc=False
)

_hist_call = functools.partial(
    pl.kernel,
    out_type=jax.ShapeDtypeStruct((NW, NPAD), jnp.float32),
    mesh=_mesh,
    compiler_params=_sc_params,
    scratch_types=[
        pltpu.VMEM((EPW,), jnp.int32),
        pltpu.VMEM((NPAD,), jnp.float32),
    ],
)(_hist_body)


def _h_body(x_ref, w_ref, h_ref):
    h_ref[...] = jnp.dot(
        x_ref[...], w_ref[...], preferred_element_type=jnp.float32
    )


def _h_call(x_pad, w_pad):
    return pl.pallas_call(
        _h_body,
        out_shape=jax.ShapeDtypeStruct((NPAD, CPAD), jnp.float32),
    )(x_pad, w_pad)


def _g_body(h_ref, hist_ref, g_ref):
    ones = jnp.ones((NW, 1), jnp.float32)
    deg = lax.dot_general(
        hist_ref[...], ones,
        dimension_numbers=(((0,), (0,)), ((), ())),
        preferred_element_type=jnp.float32,
    ) + 1.0
    dinv = lax.rsqrt(deg)
    g_ref[...] = h_ref[...] * dinv


def _g_call(h, hist):
    return pl.pallas_call(
        _g_body,
        out_shape=jax.ShapeDtypeStruct((NPAD, CPAD), jnp.float32),
    )(h, hist)


def _main_body(g_hbm, src_hbm, dst_hbm, zero_hbm, out_hbm,
               src_v, dst_v, rows0, rows1, g_sh, acc_sh, sem0, sem1):
    c = lax.axis_index("c")
    s = lax.axis_index("s")
    wid = s * NUM_SC + c
    pltpu.sync_copy(zero_hbm.at[pl.ds(s * RPS, RPS)],
                    acc_sh.at[pl.ds(s * RPS, RPS)])
    pltpu.sync_copy(g_hbm.at[pl.ds(s * RPS, RPS)],
                    g_sh.at[pl.ds(s * RPS, RPS)])
    pltpu.sync_copy(src_hbm.at[wid], src_v)
    pltpu.sync_copy(dst_hbm.at[wid], dst_v)
    plsc.subcore_barrier()

    pltpu.async_copy(g_sh.at[src_v.at[0]], rows0, sem0)
    pltpu.async_copy(g_sh.at[src_v.at[1]], rows1, sem1)

    @pl.loop(0, EB, step=2)
    def _(j):
        pltpu.make_async_copy(g_sh.at[src_v.at[0]], rows0, sem0).wait()
        pltpu.sync_copy(rows0, acc_sh.at[dst_v.at[j]], add=True)

        @pl.when(j + 2 < EB)
        def _():
            pltpu.async_copy(g_sh.at[src_v.at[j + 2]], rows0, sem0)

        pltpu.make_async_copy(g_sh.at[src_v.at[1]], rows1, sem1).wait()
        pltpu.sync_copy(rows1, acc_sh.at[dst_v.at[j + 1]], add=True)

        @pl.when(j + 3 < EB)
        def _():
            pltpu.async_copy(g_sh.at[src_v.at[j + 3]], rows1, sem1)

    plsc.subcore_barrier()
    pltpu.sync_copy(acc_sh.at[pl.ds(s * RPS, RPS)],
                    out_hbm.at[c, pl.ds(s * RPS, RPS)])


_main_call = functools.partial(
    pl.kernel,
    out_type=jax.ShapeDtypeStruct((NUM_SC, NPAD, CPAD), jnp.float32),
    mesh=_mesh,
    compiler_params=_sc_params,
    scratch_types=[
        pltpu.VMEM((EB, BLK), jnp.int32),
        pltpu.VMEM((EB, BLK), jnp.int32),
        pltpu.VMEM((BLK, CPAD), jnp.float32),
        pltpu.VMEM((BLK, CPAD), jnp.float32),
        pltpu.VMEM_SHARED((NPAD, CPAD), jnp.float32),
        pltpu.VMEM_SHARED((NPAD, CPAD), jnp.float32),
        pltpu.SemaphoreType.DMA,
        pltpu.SemaphoreType.DMA,
    ],
)(_main_body)


def _final_body(acc_ref, g_ref, hist_ref, b_ref, o_ref):
    ones = jnp.ones((NW, 1), jnp.float32)
    deg = lax.dot_general(
        hist_ref[...], ones,
        dimension_numbers=(((0,), (0,)), ((), ())),
        preferred_element_type=jnp.float32,
    ) + 1.0
    dinv = lax.rsqrt(deg)
    total = acc_ref[0] + acc_ref[1] + g_ref[...]
    z = total * dinv + b_ref[...]
    m = jnp.max(z, axis=1, keepdims=True)
    lse = jnp.log(jnp.sum(jnp.exp(z - m), axis=1, keepdims=True))
    o_ref[...] = (z - m - lse)[:N, :NCLS]


def _final_call(acc, g, hist, b_pad):
    return pl.pallas_call(
        _final_body,
        out_shape=jax.ShapeDtypeStruct((N, NCLS), jnp.float32),
    )(acc, g, hist, b_pad)


@jax.jit
def kernel(x, edge_index, W, b):
    src = edge_index[0].astype(jnp.int32)
    dst = edge_index[1].astype(jnp.int32)
    pad_src = jnp.full((EPAD - E,), N, jnp.int32)
    pad_dst = N + jnp.arange(EPAD - E, dtype=jnp.int32) % (NPAD - N)
    src_p = jnp.concatenate([src, pad_src]).reshape(NW, EB, BLK)
    dst_p = jnp.concatenate([dst, pad_dst]).reshape(NW, EB, BLK)
    dst_flat = dst_p.reshape(NW, EPW)

    x_pad = jnp.pad(x.astype(jnp.float32), ((0, NPAD - N), (0, 0)))
    w_pad = jnp.pad(W.astype(jnp.float32), ((0, 0), (0, CPAD - NCLS)))
    b_pad = jnp.full((1, CPAD), -1e30, jnp.float32).at[0, :NCLS].set(b)
    zeros = jnp.zeros((NPAD, CPAD), jnp.float32)

    hist = _hist_call(dst_flat)
    h = _h_call(x_pad, w_pad)
    g = _g_call(h, hist)
    acc = _main_call(g, src_p, dst_p, zeros)
    return _final_call(acc, g, hist, b_pad)


# speedup vs baseline: 57.7999x; 1.1094x over previous
import dataclasses
import functools

import jax
import jax.numpy as jnp
from jax import lax
from jax.experimental import pallas as pl
from jax.experimental.pallas import tpu as pltpu
from jax.experimental.pallas import tpu_sc as plsc

N = 10000
E = 320000
NF = 128
NCLS = 40

NUM_SC = 2
NUM_SUB = 16
NW = NUM_SC * NUM_SUB

NPAD = 10112
CPAD = 48
BLK = 128
EPW = E // NW
EBF = EPW // BLK
TAIL = EPW - EBF * BLK
RPS = NPAD // NUM_SUB

_mesh = plsc.VectorSubcoreMesh(
    core_axis_name="c", subcore_axis_name="s",
    num_cores=NUM_SC, num_subcores=NUM_SUB,
)


def _hist_body(ei_hbm, out_hbm, idx_v, hist_v):
    wid = lax.axis_index("s") * NUM_SC + lax.axis_index("c")
    pltpu.sync_copy(ei_hbm.at[1, pl.ds(wid * EPW, EPW)], idx_v)
    zeros16 = jnp.zeros((16,), jnp.float32)

    @pl.loop(0, NPAD // 16)
    def _(i):
        hist_v[pl.ds(i * 16, 16)] = zeros16

    ones16 = jnp.ones((16,), jnp.float32)

    @pl.loop(0, EPW // 16)
    def _(i):
        idx = idx_v[pl.ds(i * 16, 16)]
        plsc.addupdate_scatter(hist_v, [idx], ones16)

    pltpu.sync_copy(hist_v, out_hbm.at[wid])


_sc_params = pltpu.CompilerParams(
    needs_layout_passes=False, use_tc_tiling_on_sc=False
)

_hist_call = functools.partial(
    pl.kernel,
    out_type=jax.ShapeDtypeStruct((NW, NPAD), jnp.float32),
    mesh=_mesh,
    compiler_params=_sc_params,
    scratch_types=[
        pltpu.VMEM((EPW,), jnp.int32),
        pltpu.VMEM((NPAD,), jnp.float32),
    ],
)(_hist_body)


def _h_body(x_ref, w_ref, h_ref):
    h = jnp.dot(x_ref[...], w_ref[...], preferred_element_type=jnp.float32)
    h_ref[...] = jnp.pad(h, ((0, NPAD - N), (0, CPAD - NCLS)))


def _h_call(x, w):
    return pl.pallas_call(
        _h_body,
        out_shape=jax.ShapeDtypeStruct((NPAD, CPAD), jnp.float32),
    )(x, w)


def _g_body(h_ref, hist_ref, g_ref):
    ones = jnp.ones((NW, 1), jnp.float32)
    deg = lax.dot_general(
        hist_ref[...], ones,
        dimension_numbers=(((0,), (0,)), ((), ())),
        preferred_element_type=jnp.float32,
    ) + 1.0
    dinv = lax.rsqrt(deg)
    g_ref[...] = h_ref[...] * dinv


def _g_call(h, hist):
    return pl.pallas_call(
        _g_body,
        out_shape=jax.ShapeDtypeStruct((NPAD, CPAD), jnp.float32),
    )(h, hist)


def _blk(v_ref, j):
    return v_ref.at[pl.ds(pl.multiple_of(j * BLK, BLK), BLK)]


def _main_body(g_hbm, ei_hbm, zero_hbm, out_hbm,
               src_v, dst_v, rows0, rows1, g_sh, acc_sh, sem0, sem1):
    c = lax.axis_index("c")
    s = lax.axis_index("s")
    wid = s * NUM_SC + c
    base = wid * EPW
    pltpu.sync_copy(zero_hbm.at[pl.ds(s * RPS, RPS)],
                    acc_sh.at[pl.ds(s * RPS, RPS)])
    pltpu.sync_copy(g_hbm.at[pl.ds(s * RPS, RPS)],
                    g_sh.at[pl.ds(s * RPS, RPS)])
    pltpu.sync_copy(ei_hbm.at[0, pl.ds(base, EPW)], src_v)
    pltpu.sync_copy(ei_hbm.at[1, pl.ds(base, EPW)], dst_v)
    plsc.subcore_barrier()

    pltpu.async_copy(g_sh.at[_blk(src_v, 0)], rows0, sem0)
    pltpu.async_copy(g_sh.at[_blk(src_v, 1)], rows1, sem1)

    @pl.loop(0, EBF, step=2)
    def _(j):
        pltpu.make_async_copy(g_sh.at[_blk(src_v, 0)], rows0, sem0).wait()
        pltpu.sync_copy(rows0, acc_sh.at[_blk(dst_v, j)], add=True)

        @pl.when(j + 2 < EBF)
        def _():
            pltpu.async_copy(g_sh.at[_blk(src_v, j + 2)], rows0, sem0)

        pltpu.make_async_copy(g_sh.at[_blk(src_v, 1)], rows1, sem1).wait()
        pltpu.sync_copy(rows1, acc_sh.at[_blk(dst_v, j + 1)], add=True)

        @pl.when(j + 3 < EBF)
        def _():
            pltpu.async_copy(g_sh.at[_blk(src_v, j + 3)], rows1, sem1)

    tail = pl.ds(EBF * BLK, TAIL)
    pltpu.sync_copy(g_sh.at[src_v.at[tail]], rows0.at[pl.ds(0, TAIL)])
    pltpu.sync_copy(rows0.at[pl.ds(0, TAIL)], acc_sh.at[dst_v.at[tail]],
                    add=True)

    plsc.subcore_barrier()
    pltpu.sync_copy(acc_sh.at[pl.ds(s * RPS, RPS)],
                    out_hbm.at[c, pl.ds(s * RPS, RPS)])


_main_call = functools.partial(
    pl.kernel,
    out_type=jax.ShapeDtypeStruct((NUM_SC, NPAD, CPAD), jnp.float32),
    mesh=_mesh,
    compiler_params=_sc_params,
    scratch_types=[
        pltpu.VMEM((EPW,), jnp.int32),
        pltpu.VMEM((EPW,), jnp.int32),
        pltpu.VMEM((BLK, CPAD), jnp.float32),
        pltpu.VMEM((BLK, CPAD), jnp.float32),
        pltpu.VMEM_SHARED((NPAD, CPAD), jnp.float32),
        pltpu.VMEM_SHARED((NPAD, CPAD), jnp.float32),
        pltpu.SemaphoreType.DMA,
        pltpu.SemaphoreType.DMA,
    ],
)(_main_body)


def _final_body(acc_ref, g_ref, hist_ref, b_ref, o_ref):
    ones = jnp.ones((NW, 1), jnp.float32)
    deg = lax.dot_general(
        hist_ref[...], ones,
        dimension_numbers=(((0,), (0,)), ((), ())),
        preferred_element_type=jnp.float32,
    ) + 1.0
    dinv = lax.rsqrt(deg)
    total = acc_ref[0] + acc_ref[1] + g_ref[...]
    z = total[:, :NCLS] * dinv + b_ref[...]
    m = jnp.max(z, axis=1, keepdims=True)
    lse = jnp.log(jnp.sum(jnp.exp(z - m), axis=1, keepdims=True))
    o_ref[...] = (z - m - lse)[:N]


def _final_call(acc, g, hist, b):
    return pl.pallas_call(
        _final_body,
        out_shape=jax.ShapeDtypeStruct((N, NCLS), jnp.float32),
    )(acc, g, hist, b)


@jax.jit
def kernel(x, edge_index, W, b):
    ei = edge_index.astype(jnp.int32)
    zeros = jnp.zeros((NPAD, CPAD), jnp.float32)

    hist = _hist_call(ei)
    h = _h_call(x.astype(jnp.float32), W.astype(jnp.float32))
    g = _g_call(h, hist)
    acc = _main_call(g, ei, zeros)
    return _final_call(acc, g, hist, b.reshape(1, NCLS))
